# Initial kernel scaffold; baseline (speedup 1.0000x reference)
import jax
import jax.numpy as jnp
from jax.experimental import pallas as pl


def kernel(x, Wg, W1, W2, W3):
    raise NotImplementedError("write your pallas kernel here")


# speedup vs baseline: 2.0772x; 2.0772x over previous
import functools

import jax
import jax.numpy as jnp
from jax.experimental import pallas as pl
from jax.experimental.pallas import tpu as pltpu

E = 8
K = 2
D = 2048
F = 8192
T = 4096
P = T * K
BLK_T = 256
NB = P // BLK_T + E
NP = NB * BLK_T
F_BLK1 = 1024
NF1 = F // F_BLK1
F_BLK2 = 2048
NF2 = F // F_BLK2


def _ffn1_body(be_ref, xs_ref, w1_ref, w3_ref, ws_ref, h_ref):
    x = xs_ref[...]
    a = jax.lax.dot_general(x, w1_ref[0], (((1,), (1,)), ((), ())),
                            preferred_element_type=jnp.float32)
    b = jax.lax.dot_general(x, w3_ref[0], (((1,), (1,)), ((), ())),
                            preferred_element_type=jnp.float32)
    h_ref[...] = (a * jax.nn.sigmoid(a)) * b * ws_ref[...]


def _ffn2_body(be_ref, h_ref, w2_ref, y_ref):
    y_ref[0] = jax.lax.dot_general(h_ref[...], w2_ref[0],
                                   (((1,), (1,)), ((), ())),
                                   preferred_element_type=jnp.float32)


def _grouped_ffn(xs, ws, be, W1, W2, W3):
    h = pl.pallas_call(
        _ffn1_body,
        grid_spec=pltpu.PrefetchScalarGridSpec(
            num_scalar_prefetch=1,
            grid=(NF1, NB),
            in_specs=[
                pl.BlockSpec((BLK_T, D), lambda f, b, be: (b, 0)),
                pl.BlockSpec((1, F_BLK1, D), lambda f, b, be: (be[b], f, 0)),
                pl.BlockSpec((1, F_BLK1, D), lambda f, b, be: (be[b], f, 0)),
                pl.BlockSpec((BLK_T, 1), lambda f, b, be: (b, 0)),
            ],
            out_specs=pl.BlockSpec((BLK_T, F_BLK1), lambda f, b, be: (b, f)),
        ),
        out_shape=jax.ShapeDtypeStruct((NP, F), jnp.float32),
    )(be, xs, W1, W3, ws)

    ys_part = pl.pallas_call(
        _ffn2_body,
        grid_spec=pltpu.PrefetchScalarGridSpec(
            num_scalar_prefetch=1,
            grid=(NF2, NB),
            in_specs=[
                pl.BlockSpec((BLK_T, F_BLK2), lambda f, b, be: (b, f)),
                pl.BlockSpec((1, D, F_BLK2), lambda f, b, be: (be[b], 0, f)),
            ],
            out_specs=pl.BlockSpec((1, BLK_T, D), lambda f, b, be: (f, b, 0)),
        ),
        out_shape=jax.ShapeDtypeStruct((NF2, NP, D), jnp.float32),
    )(be, h, W2)
    return ys_part


def kernel(x, Wg, W1, W2, W3):
    bs, sq, dim = x.shape
    xf = x.reshape(-1, dim)

    router_logits = jnp.dot(xf, Wg.T)
    probs = jax.nn.softmax(router_logits.astype(jnp.float32), axis=-1)
    router_weights, chosen_expert = jax.lax.top_k(probs, K)
    router_weights = router_weights / jnp.sum(router_weights, axis=-1,
                                              keepdims=True)
    router_weights = router_weights.astype(x.dtype)

    e_flat = chosen_expert.reshape(-1).astype(jnp.int32)
    w_flat = router_weights.reshape(-1)
    tok_flat = jnp.arange(P, dtype=jnp.int32) // K
    oh = (e_flat[:, None] == jnp.arange(E, dtype=jnp.int32)[None, :])
    cum = jnp.cumsum(oh.astype(jnp.int32), axis=0)
    rank = jnp.take_along_axis(cum, e_flat[:, None], axis=1)[:, 0] - 1
    counts = cum[-1]
    blocks_per_e = (counts + BLK_T - 1) // BLK_T
    off = BLK_T * (jnp.cumsum(blocks_per_e) - blocks_per_e)
    dst = off[e_flat] + rank
    tok_sorted = jnp.zeros((NP,), jnp.int32).at[dst].set(tok_flat)
    ws = jnp.zeros((NP,), jnp.float32).at[dst].set(w_flat).reshape(NP, 1)
    starts = (off // BLK_T).astype(jnp.int32)
    bidx = jnp.arange(NB, dtype=jnp.int32)
    be = (jnp.searchsorted(starts, bidx, side="right") - 1).astype(jnp.int32)

    xs = xf[tok_sorted]

    ys_part = _grouped_ffn(xs, ws, be, W1, W2, W3)

    slots = dst.reshape(T, K)
    ysum = ys_part.sum(axis=0)
    final = ysum[slots[:, 0]] + ysum[slots[:, 1]]
    return final.reshape(bs, sq, dim), probs


# speedup vs baseline: 2.0980x; 1.0100x over previous
import functools

import jax
import jax.numpy as jnp
from jax.experimental import pallas as pl
from jax.experimental.pallas import tpu as pltpu

E = 8
K = 2
D = 2048
F = 8192
T = 4096
P = T * K
BLK_T = 256
NB = P // BLK_T + E
NP = NB * BLK_T
F_BLK1 = 1024
NF1 = F // F_BLK1
F_BLK2 = 2048
NF2 = F // F_BLK2


def _ffn1_body(be_ref, xs_ref, w1_ref, w3_ref, ws_ref, h_ref):
    x = xs_ref[...].astype(jnp.bfloat16)
    a = jax.lax.dot_general(x, w1_ref[0].astype(jnp.bfloat16),
                            (((1,), (1,)), ((), ())),
                            preferred_element_type=jnp.float32)
    b = jax.lax.dot_general(x, w3_ref[0].astype(jnp.bfloat16),
                            (((1,), (1,)), ((), ())),
                            preferred_element_type=jnp.float32)
    h_ref[...] = ((a * jax.nn.sigmoid(a)) * b
                  * ws_ref[...]).astype(jnp.bfloat16)


def _ffn2_body(be_ref, h_ref, w2_ref, y_ref):
    y_ref[0] = jax.lax.dot_general(h_ref[...],
                                   w2_ref[0].astype(jnp.bfloat16),
                                   (((1,), (1,)), ((), ())),
                                   preferred_element_type=jnp.float32)


def _grouped_ffn(xs, ws, be, W1, W2, W3):
    h = pl.pallas_call(
        _ffn1_body,
        grid_spec=pltpu.PrefetchScalarGridSpec(
            num_scalar_prefetch=1,
            grid=(NF1, NB),
            in_specs=[
                pl.BlockSpec((BLK_T, D), lambda f, b, be: (b, 0)),
                pl.BlockSpec((1, F_BLK1, D), lambda f, b, be: (be[b], f, 0)),
                pl.BlockSpec((1, F_BLK1, D), lambda f, b, be: (be[b], f, 0)),
                pl.BlockSpec((BLK_T, 1), lambda f, b, be: (b, 0)),
            ],
            out_specs=pl.BlockSpec((BLK_T, F_BLK1), lambda f, b, be: (b, f)),
        ),
        out_shape=jax.ShapeDtypeStruct((NP, F), jnp.bfloat16),
    )(be, xs, W1, W3, ws)

    ys_part = pl.pallas_call(
        _ffn2_body,
        grid_spec=pltpu.PrefetchScalarGridSpec(
            num_scalar_prefetch=1,
            grid=(NF2, NB),
            in_specs=[
                pl.BlockSpec((BLK_T, F_BLK2), lambda f, b, be: (b, f)),
                pl.BlockSpec((1, D, F_BLK2), lambda f, b, be: (be[b], 0, f)),
            ],
            out_specs=pl.BlockSpec((1, BLK_T, D), lambda f, b, be: (f, b, 0)),
        ),
        out_shape=jax.ShapeDtypeStruct((NF2, NP, D), jnp.float32),
    )(be, h, W2)
    return ys_part


def kernel(x, Wg, W1, W2, W3):
    bs, sq, dim = x.shape
    xf = x.reshape(-1, dim)

    router_logits = jnp.dot(xf, Wg.T)
    probs = jax.nn.softmax(router_logits.astype(jnp.float32), axis=-1)
    router_weights, chosen_expert = jax.lax.top_k(probs, K)
    router_weights = router_weights / jnp.sum(router_weights, axis=-1,
                                              keepdims=True)
    router_weights = router_weights.astype(x.dtype)

    e_flat = chosen_expert.reshape(-1).astype(jnp.int32)
    w_flat = router_weights.reshape(-1)
    tok_flat = jnp.arange(P, dtype=jnp.int32) // K
    oh = (e_flat[:, None] == jnp.arange(E, dtype=jnp.int32)[None, :])
    cum = jnp.cumsum(oh.astype(jnp.int32), axis=0)
    rank = jnp.take_along_axis(cum, e_flat[:, None], axis=1)[:, 0] - 1
    counts = cum[-1]
    blocks_per_e = (counts + BLK_T - 1) // BLK_T
    off = BLK_T * (jnp.cumsum(blocks_per_e) - blocks_per_e)
    dst = off[e_flat] + rank
    tok_sorted = jnp.zeros((NP,), jnp.int32).at[dst].set(tok_flat)
    ws = jnp.zeros((NP,), jnp.float32).at[dst].set(w_flat).reshape(NP, 1)
    starts = (off // BLK_T).astype(jnp.int32)
    bidx = jnp.arange(NB, dtype=jnp.int32)
    be = (jnp.searchsorted(starts, bidx, side="right") - 1).astype(jnp.int32)

    xs = xf[tok_sorted]

    ys_part = _grouped_ffn(xs, ws, be, W1, W2, W3)

    slots = dst.reshape(T, K)
    ysum = ys_part.sum(axis=0)
    final = ysum[slots[:, 0]] + ysum[slots[:, 1]]
    return final.reshape(bs, sq, dim), probs
